# baseline (speedup 1.0000x reference)
import functools

import jax
import jax.numpy as jnp
from jax import lax
from jax.experimental import pallas as pl
from jax.experimental.pallas import tpu as pltpu
from jax.experimental.pallas import tpu_sc as plsc

N = 2048
DIM = 128
BLK = 256
NBLK = N // BLK
TOPK = 10
HALF = 5
KPAD = 16


def _dense_body(s_all_ref, s_blk_ref, t_all_ref, t_blk_ref, idx_ref,
                idx_blk_ref, loss_ref, d_ref, mean_ref, topk_ref):
    i = pl.program_id(0)
    s_all = s_all_ref[...]
    s_blk = s_blk_ref[...]
    ssq_all = jnp.sum(s_all * s_all, axis=1)
    ssq_blk = jnp.sum(s_blk * s_blk, axis=1)
    d2 = ssq_blk[:, None] + ssq_all[None, :] - 2.0 * jax.lax.dot_general(
        s_blk, s_all, (((1,), (1,)), ((), ())),
        preferred_element_type=jnp.float32)
    d2 = jnp.maximum(d2, 0.0)
    safe = jnp.where(d2 > 0.0, d2, 1.0)
    D = jnp.where(d2 > 0.0, jnp.sqrt(safe), 0.0)
    mean_i = jnp.mean(D, axis=1)
    S = D / mean_i[:, None]

    t_all = t_all_ref[...]
    t_blk = t_blk_ref[...]
    tn_all = t_all / jnp.maximum(
        jnp.sqrt(jnp.sum(t_all * t_all, axis=1, keepdims=True)), 1e-12)
    tn_blk = t_blk / jnp.maximum(
        jnp.sqrt(jnp.sum(t_blk * t_blk, axis=1, keepdims=True)), 1e-12)
    tsq_all = jnp.sum(tn_all * tn_all, axis=1)
    tsq_blk = jnp.sum(tn_blk * tn_blk, axis=1)
    td2 = tsq_blk[:, None] + tsq_all[None, :] - 2.0 * jax.lax.dot_general(
        tn_blk, tn_all, (((1,), (1,)), ((), ())),
        preferred_element_type=jnp.float32)
    td2 = jnp.maximum(td2, 0.0)
    tsafe = jnp.where(td2 > 0.0, td2, 1.0)
    Td = jnp.where(td2 > 0.0, jnp.sqrt(tsafe), 0.0)
    W_P = jnp.exp(-(Td * Td))

    idx_all = idx_ref[0, :]
    idx_blk = idx_blk_ref[0, :]
    same = idx_blk[:, None] == idx_all[None, :]
    wp = jnp.where(same, 1.0, W_P)

    col = lax.broadcasted_iota(jnp.int32, (BLK, N), 1)
    tk = []
    for _ in range(TOPK):
        m = jnp.max(wp, axis=1, keepdims=True)
        cand = jnp.min(jnp.where(wp >= m, col, N), axis=1)
        tk.append(cand)
        wp = jnp.where(col == cand[:, None], -jnp.inf, wp)
    tk_mat = jnp.stack(tk, axis=1)
    topk_ref[...] = jnp.pad(tk_mat, ((0, 0), (0, KPAD - TOPK)))

    base = jnp.maximum(1.0 - S, 0.0) ** 2
    g = jnp.maximum(S, 0.0) ** 2 - base
    row = lax.broadcasted_iota(jnp.int32, (BLK, N), 0)
    offdiag = (col != row + i * BLK).astype(jnp.float32)
    part = jnp.sum(offdiag * (base + 0.5 * W_P * g))
    lane = lax.broadcasted_iota(jnp.int32, (1, 1, 128), 2)
    loss_ref[...] = jnp.where(lane == 0, part, 0.0)

    d_ref[...] = D
    mean_ref[...] = mean_i[None, None, :]


def _dense_phase(s_emb, t_emb, idx):
    idx2 = idx.reshape(1, N)
    out = pl.pallas_call(
        _dense_body,
        grid=(NBLK,),
        in_specs=[
            pl.BlockSpec((N, DIM), lambda i: (0, 0)),
            pl.BlockSpec((BLK, DIM), lambda i: (i, 0)),
            pl.BlockSpec((N, DIM), lambda i: (0, 0)),
            pl.BlockSpec((BLK, DIM), lambda i: (i, 0)),
            pl.BlockSpec((1, N), lambda i: (0, 0)),
            pl.BlockSpec((1, BLK), lambda i: (0, i)),
        ],
        out_specs=[
            pl.BlockSpec((1, 1, 128), lambda i: (i, 0, 0)),
            pl.BlockSpec((BLK, N), lambda i: (i, 0)),
            pl.BlockSpec((1, 1, BLK), lambda i: (i, 0, 0)),
            pl.BlockSpec((BLK, KPAD), lambda i: (i, 0)),
        ],
        out_shape=[
            jax.ShapeDtypeStruct((NBLK, 1, 128), jnp.float32),
            jax.ShapeDtypeStruct((N, N), jnp.float32),
            jax.ShapeDtypeStruct((NBLK, 1, BLK), jnp.float32),
            jax.ShapeDtypeStruct((N, KPAD), jnp.int32),
        ],
    )(s_emb, s_emb, t_emb, t_emb, idx2, idx2)
    return out


_NC = 2
_NS = 16
_NW = _NC * _NS
_ROWS_PER_TEC = N // _NW
_ROWS_PER_SID = N // _NS


def _sc_body(t_hbm, means_hbm, d_hbm, out_hbm, mut_hbm, wc_hbm,
             T_v, mut_v, wc_v, means_v, invc_v, drow_v, acc_v, sem):
    cid = lax.axis_index("c")
    sid = lax.axis_index("s")
    wid = sid * _NC + cid
    lanes = lax.iota(jnp.int32, 16)
    kvalid = lanes < TOPK
    zi = jnp.zeros((16,), jnp.int32)

    pltpu.sync_copy(t_hbm, T_v)
    pltpu.sync_copy(means_hbm, means_v)

    base_a = sid * _ROWS_PER_SID

    def mut_row(r, _):
        jv = plsc.load_gather(T_v, [r * KPAD + lanes])
        macc = jnp.zeros((16,), jnp.int32)
        for kp in range(TOPK):
            tv = plsc.load_gather(T_v, [jv * KPAD + kp], mask=kvalid)
            macc = macc | jnp.where((tv == r) & kvalid, 1, 0)
        plsc.store_scatter(mut_v, [r * KPAD + lanes], macc)
        cnt = jnp.sum(macc.astype(jnp.float32), axis=0)
        invv = 1.0 / jnp.maximum(jnp.zeros((16,), jnp.float32) + cnt, 1.0)
        plsc.store_scatter(invc_v, [zi + (r - base_a)], invv,
                           mask=lanes == 0)
        return 0

    lax.fori_loop(base_a, base_a + _ROWS_PER_SID, mut_row, 0)

    off_a = base_a * KPAD
    sz_a = _ROWS_PER_SID * KPAD
    coff = cid * N * KPAD
    pltpu.sync_copy(mut_v.at[pl.ds(off_a, sz_a)],
                    mut_hbm.at[pl.ds(coff + off_a, sz_a)])
    plsc.subcore_barrier()
    pltpu.sync_copy(mut_hbm.at[pl.ds(coff, N * KPAD)], mut_v)

    def wc_row(r, _):
        jv = plsc.load_gather(T_v, [r * KPAD + lanes])
        mutr = plsc.load_gather(mut_v, [r * KPAD + lanes])
        act = kvalid & (mutr != 0)
        Mv = jnp.zeros((16,), jnp.float32)
        for b in range(TOPK):
            tb = plsc.load_gather(T_v, [jv * KPAD + b], mask=act)
            mb = plsc.load_gather(mut_v, [jv * KPAD + b], mask=act)
            member = jnp.zeros((16,), jnp.int32)
            for a in range(TOPK):
                hit = (tb == jv[a]) & (mutr[a] != 0)
                member = member | jnp.where(hit, 1, 0)
            Mv = Mv + jnp.where((member != 0) & (mb != 0), 1.0, 0.0)
        invr = plsc.load_gather(invc_v, [zi + (r - base_a)])
        wcval = jnp.where(act, Mv * invr, -1.0)
        plsc.store_scatter(wc_v, [r * KPAD + lanes], wcval)
        return 0

    lax.fori_loop(base_a, base_a + _ROWS_PER_SID, wc_row, 0)

    pltpu.sync_copy(wc_v.at[pl.ds(off_a, sz_a)],
                    wc_hbm.at[pl.ds(coff + off_a, sz_a)])
    plsc.subcore_barrier()
    pltpu.sync_copy(wc_hbm.at[pl.ds(coff, N * KPAD)], wc_v)

    base_b = wid * _ROWS_PER_TEC

    def row_i(i, acc):
        par = lax.rem(i - base_b, 2)
        pltpu.sync_copy(d_hbm.at[i], drow_v.at[par])
        miv = plsc.load_gather(means_v, [zi + i])
        Trow = T_v[pl.ds(i * KPAD, 16)]
        for m in range(HALF):
            r = Trow[m]
            rvec = zi + r
            wcv = plsc.load_gather(wc_v, [rvec * KPAD + lanes])
            jv = plsc.load_gather(T_v, [rvec * KPAD + lanes])
            valid = kvalid & (wcv >= 0.0) & (jv != i)
            dij = plsc.load_gather(drow_v, [zi + par, jv], mask=valid)
            mj = plsc.load_gather(means_v, [jv], mask=valid)
            sij = dij / miv
            sji = dij / jnp.where(valid, mj, 1.0)
            pij = jnp.maximum(sij, 0.0)
            qij = jnp.maximum(1.0 - sij, 0.0)
            pji = jnp.maximum(sji, 0.0)
            qji = jnp.maximum(1.0 - sji, 0.0)
            gij = pij * pij - qij * qij
            gji = pji * pji - qji * qji
            acc = acc + jnp.where(valid, wcv * (gij + gji), 0.0)
        return acc

    acc = lax.fori_loop(base_b, base_b + _ROWS_PER_TEC, row_i,
                        jnp.zeros((16,), jnp.float32))
    acc_v[...] = acc * (0.25 / HALF)
    pltpu.sync_copy(acc_v, out_hbm.at[pl.ds(wid * 16, 16)])


def _sparse_corr_sc(T, means, D):
    t_flat = T.reshape(N * KPAD)
    mesh = plsc.VectorSubcoreMesh(core_axis_name="c", subcore_axis_name="s")
    out, _, _ = pl.kernel(
        _sc_body,
        out_type=(
            jax.ShapeDtypeStruct((_NW * 16,), jnp.float32),
            jax.ShapeDtypeStruct((_NC * N * KPAD,), jnp.int32),
            jax.ShapeDtypeStruct((_NC * N * KPAD,), jnp.float32),
        ),
        mesh=mesh,
        compiler_params=pltpu.CompilerParams(needs_layout_passes=False),
        scratch_types=[
            pltpu.VMEM((N * KPAD,), jnp.int32),
            pltpu.VMEM((N * KPAD,), jnp.int32),
            pltpu.VMEM((N * KPAD,), jnp.float32),
            pltpu.VMEM((N,), jnp.float32),
            pltpu.VMEM((_ROWS_PER_SID,), jnp.float32),
            pltpu.VMEM((2, N), jnp.float32),
            pltpu.VMEM((16,), jnp.float32),
            pltpu.SemaphoreType.DMA,
        ],
    )(t_flat, means, D)
    return jnp.sum(out)


def _sparse_corr_jnp(T, means, D):
    Tk = T[:, :TOPK]
    rows = jnp.arange(N)
    mut = (Tk[Tk] == rows[:, None, None]).any(-1)
    cnt = mut.sum(1).astype(jnp.float32)
    invc = 1.0 / jnp.maximum(cnt, 1.0)

    def g_pair(i, j):
        dij = D[i, j]
        sij = dij / means[i]
        sji = dij / means[j]
        gij = jnp.maximum(sij, 0.0) ** 2 - jnp.maximum(1.0 - sij, 0.0) ** 2
        gji = jnp.maximum(sji, 0.0) ** 2 - jnp.maximum(1.0 - sji, 0.0) ** 2
        return gij + gji

    r = Tk[:, :HALF]
    jmat = Tk[r]
    mut_r = mut[r]
    valid = mut_r & (jmat != rows[:, None, None])
    Tr = Tk[r]
    mutr = mut[r]
    Tj = Tk[jmat]
    mutj = mut[jmat]
    match = ((Tr[:, :, None, :, None] == Tj[:, :, :, None, :])
             & mutr[:, :, None, :, None] & mutj[:, :, :, None, :])
    M = match.sum((-1, -2)).astype(jnp.float32)
    coeff = M * invc[r][:, :, None]
    gp = g_pair(rows[:, None, None], jmat)
    corr = jnp.where(valid, coeff * gp, 0.0).sum()
    return 0.25 / HALF * corr


def kernel(s_emb, t_emb, idx):
    loss_parts, D, means, T = _dense_phase(s_emb, t_emb, idx)
    means = means.reshape(N)
    corr = _sparse_corr_sc(T, means, D)
    return (jnp.sum(loss_parts) + corr) / (N * (N - 1))


# speedup vs baseline: 1.0092x; 1.0092x over previous
import functools

import jax
import jax.numpy as jnp
from jax import lax
from jax.experimental import pallas as pl
from jax.experimental.pallas import tpu as pltpu
from jax.experimental.pallas import tpu_sc as plsc

N = 2048
DIM = 128
BLK = 256
NBLK = N // BLK
TOPK = 10
HALF = 5
KPAD = 17


def _dense_body(s_all_ref, s_blk_ref, t_all_ref, t_blk_ref, idx_ref,
                idx_blk_ref, loss_ref, d_ref, mean_ref, topk_ref):
    i = pl.program_id(0)
    s_all = s_all_ref[...]
    s_blk = s_blk_ref[...]
    ssq_all = jnp.sum(s_all * s_all, axis=1)
    ssq_blk = jnp.sum(s_blk * s_blk, axis=1)
    d2 = ssq_blk[:, None] + ssq_all[None, :] - 2.0 * jax.lax.dot_general(
        s_blk, s_all, (((1,), (1,)), ((), ())),
        preferred_element_type=jnp.float32)
    d2 = jnp.maximum(d2, 0.0)
    safe = jnp.where(d2 > 0.0, d2, 1.0)
    D = jnp.where(d2 > 0.0, jnp.sqrt(safe), 0.0)
    mean_i = jnp.mean(D, axis=1)
    S = D / mean_i[:, None]

    t_all = t_all_ref[...]
    t_blk = t_blk_ref[...]
    tn_all = t_all / jnp.maximum(
        jnp.sqrt(jnp.sum(t_all * t_all, axis=1, keepdims=True)), 1e-12)
    tn_blk = t_blk / jnp.maximum(
        jnp.sqrt(jnp.sum(t_blk * t_blk, axis=1, keepdims=True)), 1e-12)
    tsq_all = jnp.sum(tn_all * tn_all, axis=1)
    tsq_blk = jnp.sum(tn_blk * tn_blk, axis=1)
    td2 = tsq_blk[:, None] + tsq_all[None, :] - 2.0 * jax.lax.dot_general(
        tn_blk, tn_all, (((1,), (1,)), ((), ())),
        preferred_element_type=jnp.float32)
    td2 = jnp.maximum(td2, 0.0)
    tsafe = jnp.where(td2 > 0.0, td2, 1.0)
    Td = jnp.where(td2 > 0.0, jnp.sqrt(tsafe), 0.0)
    W_P = jnp.exp(-(Td * Td))

    idx_all = idx_ref[0, :]
    idx_blk = idx_blk_ref[0, :]
    same = idx_blk[:, None] == idx_all[None, :]
    wp = jnp.where(same, 1.0, W_P)

    col = lax.broadcasted_iota(jnp.int32, (BLK, N), 1)
    tk = []
    for _ in range(TOPK):
        m = jnp.max(wp, axis=1, keepdims=True)
        cand = jnp.min(jnp.where(wp >= m, col, N), axis=1)
        tk.append(cand)
        wp = jnp.where(col == cand[:, None], -jnp.inf, wp)
    tk_mat = jnp.stack(tk, axis=1)
    topk_ref[...] = jnp.pad(tk_mat, ((0, 0), (0, KPAD - TOPK)))

    base = jnp.maximum(1.0 - S, 0.0) ** 2
    g = jnp.maximum(S, 0.0) ** 2 - base
    row = lax.broadcasted_iota(jnp.int32, (BLK, N), 0)
    offdiag = (col != row + i * BLK).astype(jnp.float32)
    part = jnp.sum(offdiag * (base + 0.5 * W_P * g))
    lane = lax.broadcasted_iota(jnp.int32, (1, 1, 128), 2)
    loss_ref[...] = jnp.where(lane == 0, part, 0.0)

    d_ref[...] = D
    mean_ref[...] = mean_i[None, None, :]


def _dense_phase(s_emb, t_emb, idx):
    idx2 = idx.reshape(1, N)
    out = pl.pallas_call(
        _dense_body,
        grid=(NBLK,),
        in_specs=[
            pl.BlockSpec((N, DIM), lambda i: (0, 0)),
            pl.BlockSpec((BLK, DIM), lambda i: (i, 0)),
            pl.BlockSpec((N, DIM), lambda i: (0, 0)),
            pl.BlockSpec((BLK, DIM), lambda i: (i, 0)),
            pl.BlockSpec((1, N), lambda i: (0, 0)),
            pl.BlockSpec((1, BLK), lambda i: (0, i)),
        ],
        out_specs=[
            pl.BlockSpec((1, 1, 128), lambda i: (i, 0, 0)),
            pl.BlockSpec((BLK, N), lambda i: (i, 0)),
            pl.BlockSpec((1, 1, BLK), lambda i: (i, 0, 0)),
            pl.BlockSpec((BLK, KPAD), lambda i: (i, 0)),
        ],
        out_shape=[
            jax.ShapeDtypeStruct((NBLK, 1, 128), jnp.float32),
            jax.ShapeDtypeStruct((N, N), jnp.float32),
            jax.ShapeDtypeStruct((NBLK, 1, BLK), jnp.float32),
            jax.ShapeDtypeStruct((N, KPAD), jnp.int32),
        ],
    )(s_emb, s_emb, t_emb, t_emb, idx2, idx2)
    return out


_NC = 2
_NS = 16
_NW = _NC * _NS
_ROWS_PER_TEC = N // _NW
_ROWS_PER_SID = N // _NS


def _sc_body(t_hbm, means_hbm, d_hbm, out_hbm, mut_hbm, wc_hbm,
             T_v, mut_v, wc_v, means_v, invc_v, drow_v, acc_v, sem):
    cid = lax.axis_index("c")
    sid = lax.axis_index("s")
    wid = sid * _NC + cid
    lanes = lax.iota(jnp.int32, 16)
    kvalid = lanes < TOPK
    zi = jnp.zeros((16,), jnp.int32)

    pltpu.sync_copy(t_hbm, T_v)
    pltpu.sync_copy(means_hbm, means_v)

    base_a = sid * _ROWS_PER_SID

    def mut_row(r, _):
        jv = plsc.load_gather(T_v, [r * KPAD + lanes])
        macc = jnp.zeros((16,), jnp.int32)
        for kp in range(TOPK):
            tv = plsc.load_gather(T_v, [jv * KPAD + kp], mask=kvalid)
            macc = macc | jnp.where((tv == r) & kvalid, 1, 0)
        plsc.store_scatter(mut_v, [r * KPAD + lanes], macc)
        cnt = jnp.sum(macc.astype(jnp.float32), axis=0)
        invv = 1.0 / jnp.maximum(jnp.zeros((16,), jnp.float32) + cnt, 1.0)
        plsc.store_scatter(invc_v, [zi + (r - base_a)], invv,
                           mask=lanes == 0)
        return 0

    lax.fori_loop(base_a, base_a + _ROWS_PER_SID, mut_row, 0)

    off_a = base_a * KPAD
    sz_a = _ROWS_PER_SID * KPAD
    coff = cid * N * KPAD
    pltpu.sync_copy(mut_v.at[pl.ds(off_a, sz_a)],
                    mut_hbm.at[pl.ds(coff + off_a, sz_a)])
    plsc.subcore_barrier()
    pltpu.sync_copy(mut_hbm.at[pl.ds(coff, N * KPAD)], mut_v)

    def wc_row(r, _):
        jv = plsc.load_gather(T_v, [r * KPAD + lanes])
        mutr = plsc.load_gather(mut_v, [r * KPAD + lanes])
        act = kvalid & (mutr != 0)
        Mv = jnp.zeros((16,), jnp.float32)
        for b in range(TOPK):
            tb = plsc.load_gather(T_v, [jv * KPAD + b], mask=act)
            mb = plsc.load_gather(mut_v, [jv * KPAD + b], mask=act)
            member = jnp.zeros((16,), jnp.int32)
            for a in range(TOPK):
                hit = (tb == jv[a]) & (mutr[a] != 0)
                member = member | jnp.where(hit, 1, 0)
            Mv = Mv + jnp.where((member != 0) & (mb != 0), 1.0, 0.0)
        invr = plsc.load_gather(invc_v, [zi + (r - base_a)])
        wcval = jnp.where(act, Mv * invr, -1.0)
        plsc.store_scatter(wc_v, [r * KPAD + lanes], wcval)
        return 0

    lax.fori_loop(base_a, base_a + _ROWS_PER_SID, wc_row, 0)

    pltpu.sync_copy(wc_v.at[pl.ds(off_a, sz_a)],
                    wc_hbm.at[pl.ds(coff + off_a, sz_a)])
    plsc.subcore_barrier()
    pltpu.sync_copy(wc_hbm.at[pl.ds(coff, N * KPAD)], wc_v)

    base_b = wid * _ROWS_PER_TEC

    def row_i(i, acc):
        par = lax.rem(i - base_b, 2)
        pltpu.sync_copy(d_hbm.at[i], drow_v.at[par])
        miv = plsc.load_gather(means_v, [zi + i])
        Trow = plsc.load_gather(T_v, [i * KPAD + lanes])
        for m in range(HALF):
            r = Trow[m]
            rvec = zi + r
            wcv = plsc.load_gather(wc_v, [rvec * KPAD + lanes])
            jv = plsc.load_gather(T_v, [rvec * KPAD + lanes])
            valid = kvalid & (wcv >= 0.0) & (jv != i)
            dij = plsc.load_gather(drow_v, [zi + par, jv], mask=valid)
            mj = plsc.load_gather(means_v, [jv], mask=valid)
            sij = dij / miv
            sji = dij / jnp.where(valid, mj, 1.0)
            pij = jnp.maximum(sij, 0.0)
            qij = jnp.maximum(1.0 - sij, 0.0)
            pji = jnp.maximum(sji, 0.0)
            qji = jnp.maximum(1.0 - sji, 0.0)
            gij = pij * pij - qij * qij
            gji = pji * pji - qji * qji
            acc = acc + jnp.where(valid, wcv * (gij + gji), 0.0)
        return acc

    acc = lax.fori_loop(base_b, base_b + _ROWS_PER_TEC, row_i,
                        jnp.zeros((16,), jnp.float32))
    acc_v[...] = acc * (0.25 / HALF)
    pltpu.sync_copy(acc_v, out_hbm.at[pl.ds(wid * 16, 16)])


def _sparse_corr_sc(T, means, D):
    t_flat = T.reshape(N * KPAD)
    mesh = plsc.VectorSubcoreMesh(core_axis_name="c", subcore_axis_name="s")
    out, _, _ = pl.kernel(
        _sc_body,
        out_type=(
            jax.ShapeDtypeStruct((_NW * 16,), jnp.float32),
            jax.ShapeDtypeStruct((_NC * N * KPAD,), jnp.int32),
            jax.ShapeDtypeStruct((_NC * N * KPAD,), jnp.float32),
        ),
        mesh=mesh,
        compiler_params=pltpu.CompilerParams(needs_layout_passes=False),
        scratch_types=[
            pltpu.VMEM((N * KPAD,), jnp.int32),
            pltpu.VMEM((N * KPAD,), jnp.int32),
            pltpu.VMEM((N * KPAD,), jnp.float32),
            pltpu.VMEM((N,), jnp.float32),
            pltpu.VMEM((_ROWS_PER_SID,), jnp.float32),
            pltpu.VMEM((2, N), jnp.float32),
            pltpu.VMEM((16,), jnp.float32),
            pltpu.SemaphoreType.DMA,
        ],
    )(t_flat, means, D)
    return jnp.sum(out)


def _sparse_corr_jnp(T, means, D):
    Tk = T[:, :TOPK]
    rows = jnp.arange(N)
    mut = (Tk[Tk] == rows[:, None, None]).any(-1)
    cnt = mut.sum(1).astype(jnp.float32)
    invc = 1.0 / jnp.maximum(cnt, 1.0)

    def g_pair(i, j):
        dij = D[i, j]
        sij = dij / means[i]
        sji = dij / means[j]
        gij = jnp.maximum(sij, 0.0) ** 2 - jnp.maximum(1.0 - sij, 0.0) ** 2
        gji = jnp.maximum(sji, 0.0) ** 2 - jnp.maximum(1.0 - sji, 0.0) ** 2
        return gij + gji

    r = Tk[:, :HALF]
    jmat = Tk[r]
    mut_r = mut[r]
    valid = mut_r & (jmat != rows[:, None, None])
    Tr = Tk[r]
    mutr = mut[r]
    Tj = Tk[jmat]
    mutj = mut[jmat]
    match = ((Tr[:, :, None, :, None] == Tj[:, :, :, None, :])
             & mutr[:, :, None, :, None] & mutj[:, :, :, None, :])
    M = match.sum((-1, -2)).astype(jnp.float32)
    coeff = M * invc[r][:, :, None]
    gp = g_pair(rows[:, None, None], jmat)
    corr = jnp.where(valid, coeff * gp, 0.0).sum()
    return 0.25 / HALF * corr


def kernel(s_emb, t_emb, idx):
    loss_parts, D, means, T = _dense_phase(s_emb, t_emb, idx)
    means = means.reshape(N)
    corr = _sparse_corr_sc(T, means, D)
    return (jnp.sum(loss_parts) + corr) / (N * (N - 1))


# speedup vs baseline: 1.3835x; 1.3708x over previous
import functools

import jax
import jax.numpy as jnp
from jax import lax
from jax.experimental import pallas as pl
from jax.experimental.pallas import tpu as pltpu
from jax.experimental.pallas import tpu_sc as plsc

N = 2048
DIM = 128
BLK = 256
NBLK = N // BLK
TOPK = 10
HALF = 5
KPAD = 17


def _dense_body(s_all_ref, s_blk_ref, t_all_ref, t_blk_ref, idx_ref,
                idx_blk_ref, loss_ref, d_ref, mean_ref, topk_ref):
    i = pl.program_id(0)
    s_all = s_all_ref[...]
    s_blk = s_blk_ref[...]
    ssq_all = jnp.sum(s_all * s_all, axis=1)
    ssq_blk = jnp.sum(s_blk * s_blk, axis=1)
    d2 = ssq_blk[:, None] + ssq_all[None, :] - 2.0 * jax.lax.dot_general(
        s_blk, s_all, (((1,), (1,)), ((), ())),
        preferred_element_type=jnp.float32)
    d2 = jnp.maximum(d2, 0.0)
    safe = jnp.where(d2 > 0.0, d2, 1.0)
    D = jnp.where(d2 > 0.0, jnp.sqrt(safe), 0.0)
    mean_i = jnp.mean(D, axis=1)
    S = D / mean_i[:, None]

    t_all = t_all_ref[...]
    t_blk = t_blk_ref[...]
    tn_all = t_all / jnp.maximum(
        jnp.sqrt(jnp.sum(t_all * t_all, axis=1, keepdims=True)), 1e-12)
    tn_blk = t_blk / jnp.maximum(
        jnp.sqrt(jnp.sum(t_blk * t_blk, axis=1, keepdims=True)), 1e-12)
    tsq_all = jnp.sum(tn_all * tn_all, axis=1)
    tsq_blk = jnp.sum(tn_blk * tn_blk, axis=1)
    td2 = tsq_blk[:, None] + tsq_all[None, :] - 2.0 * jax.lax.dot_general(
        tn_blk, tn_all, (((1,), (1,)), ((), ())),
        preferred_element_type=jnp.float32)
    td2 = jnp.maximum(td2, 0.0)
    tsafe = jnp.where(td2 > 0.0, td2, 1.0)
    Td = jnp.where(td2 > 0.0, jnp.sqrt(tsafe), 0.0)
    W_P = jnp.exp(-(Td * Td))

    idx_all = idx_ref[0, :]
    idx_blk = idx_blk_ref[0, :]
    same = idx_blk[:, None] == idx_all[None, :]
    wp = jnp.where(same, 1.0, W_P)

    col = lax.broadcasted_iota(jnp.int32, (BLK, N), 1)
    tk = []
    for _ in range(TOPK):
        m = jnp.max(wp, axis=1, keepdims=True)
        cand = jnp.min(jnp.where(wp >= m, col, N), axis=1)
        tk.append(cand)
        wp = jnp.where(col == cand[:, None], -jnp.inf, wp)
    tk_mat = jnp.stack(tk, axis=1)
    topk_ref[...] = jnp.pad(tk_mat, ((0, 0), (0, KPAD - TOPK)))

    base = jnp.maximum(1.0 - S, 0.0) ** 2
    g = jnp.maximum(S, 0.0) ** 2 - base
    row = lax.broadcasted_iota(jnp.int32, (BLK, N), 0)
    offdiag = (col != row + i * BLK).astype(jnp.float32)
    part = jnp.sum(offdiag * (base + 0.5 * W_P * g))
    lane = lax.broadcasted_iota(jnp.int32, (1, 1, 128), 2)
    loss_ref[...] = jnp.where(lane == 0, part, 0.0)

    d_ref[...] = D
    mean_ref[...] = mean_i[None, None, :]


def _dense_phase(s_emb, t_emb, idx):
    idx2 = idx.reshape(1, N)
    out = pl.pallas_call(
        _dense_body,
        grid=(NBLK,),
        in_specs=[
            pl.BlockSpec((N, DIM), lambda i: (0, 0)),
            pl.BlockSpec((BLK, DIM), lambda i: (i, 0)),
            pl.BlockSpec((N, DIM), lambda i: (0, 0)),
            pl.BlockSpec((BLK, DIM), lambda i: (i, 0)),
            pl.BlockSpec((1, N), lambda i: (0, 0)),
            pl.BlockSpec((1, BLK), lambda i: (0, i)),
        ],
        out_specs=[
            pl.BlockSpec((1, 1, 128), lambda i: (i, 0, 0)),
            pl.BlockSpec((BLK, N), lambda i: (i, 0)),
            pl.BlockSpec((1, 1, BLK), lambda i: (i, 0, 0)),
            pl.BlockSpec((BLK, KPAD), lambda i: (i, 0)),
        ],
        out_shape=[
            jax.ShapeDtypeStruct((NBLK, 1, 128), jnp.float32),
            jax.ShapeDtypeStruct((N, N), jnp.float32),
            jax.ShapeDtypeStruct((NBLK, 1, BLK), jnp.float32),
            jax.ShapeDtypeStruct((N, KPAD), jnp.int32),
        ],
    )(s_emb, s_emb, t_emb, t_emb, idx2, idx2)
    return out


_NC = 2
_NS = 16
_NW = _NC * _NS
_ROWS_PER_TEC = N // _NW
_ROWS_PER_SID = N // _NS


def _sc_body(t_hbm, means_hbm, d_hbm, out_hbm, mut_hbm, wc_hbm,
             T_v, mut_v, means_v, invc_v, drow_v, acc_v, sem):
    cid = lax.axis_index("c")
    sid = lax.axis_index("s")
    wid = sid * _NC + cid
    lanes = lax.iota(jnp.int32, 16)
    kvalid = lanes < TOPK
    zi = jnp.zeros((16,), jnp.int32)

    pltpu.sync_copy(t_hbm, T_v)
    pltpu.sync_copy(means_hbm, means_v)

    base_a = sid * _ROWS_PER_SID

    def mut_row(r, _):
        jv = plsc.load_gather(T_v, [r * KPAD + lanes])
        macc = jnp.zeros((16,), jnp.int32)
        for kp in range(TOPK):
            tv = plsc.load_gather(T_v, [jv * KPAD + kp], mask=kvalid)
            macc = macc | jnp.where((tv == r) & kvalid, 1, 0)
        plsc.store_scatter(mut_v, [r * KPAD + lanes], macc)
        cnt = jnp.sum(macc.astype(jnp.float32), axis=0)
        invv = 1.0 / jnp.maximum(jnp.zeros((16,), jnp.float32) + cnt, 1.0)
        plsc.store_scatter(invc_v, [zi + r], invv, mask=lanes == 0)
        return 0

    lax.fori_loop(base_a, base_a + _ROWS_PER_SID, mut_row, 0)

    off_a = base_a * KPAD
    sz_a = _ROWS_PER_SID * KPAD
    coff = cid * N * KPAD
    pltpu.sync_copy(mut_v.at[pl.ds(off_a, sz_a)],
                    mut_hbm.at[pl.ds(coff + off_a, sz_a)])
    pltpu.sync_copy(invc_v.at[pl.ds(base_a, _ROWS_PER_SID)],
                    wc_hbm.at[pl.ds(cid * N + base_a, _ROWS_PER_SID)])
    plsc.subcore_barrier()
    pltpu.sync_copy(mut_hbm.at[pl.ds(coff, N * KPAD)], mut_v)
    pltpu.sync_copy(wc_hbm.at[pl.ds(cid * N, N)], invc_v)

    base_b = wid * _ROWS_PER_TEC

    def row_i(i, acc):
        par = lax.rem(i - base_b, 2)
        pltpu.sync_copy(d_hbm.at[i], drow_v.at[par])
        miv = plsc.load_gather(means_v, [zi + i])
        Trow = plsc.load_gather(T_v, [i * KPAD + lanes])
        for m in range(HALF):
            r = Trow[m]
            rvec = zi + r
            invr = plsc.load_gather(invc_v, [rvec])
            jv = plsc.load_gather(T_v, [rvec * KPAD + lanes])
            mutr = plsc.load_gather(mut_v, [rvec * KPAD + lanes])
            valid = kvalid & (mutr != 0) & (jv != i)
            Mv = jnp.zeros((16,), jnp.float32)
            for b in range(TOPK):
                tb = plsc.load_gather(T_v, [jv * KPAD + b], mask=valid)
                mb = plsc.load_gather(mut_v, [jv * KPAD + b], mask=valid)
                member = jnp.zeros((16,), jnp.int32)
                for a in range(TOPK):
                    hit = (tb == jv[a]) & (mutr[a] != 0)
                    member = member | jnp.where(hit, 1, 0)
                Mv = Mv + jnp.where((member != 0) & (mb != 0), 1.0, 0.0)
            dij = plsc.load_gather(drow_v, [zi + par, jv], mask=valid)
            mj = plsc.load_gather(means_v, [jv], mask=valid)
            sij = dij / miv
            sji = dij / jnp.where(valid, mj, 1.0)
            pij = jnp.maximum(sij, 0.0)
            qij = jnp.maximum(1.0 - sij, 0.0)
            pji = jnp.maximum(sji, 0.0)
            qji = jnp.maximum(1.0 - sji, 0.0)
            gij = pij * pij - qij * qij
            gji = pji * pji - qji * qji
            acc = acc + jnp.where(valid, Mv * invr * (gij + gji), 0.0)
        return acc

    acc = lax.fori_loop(base_b, base_b + _ROWS_PER_TEC, row_i,
                        jnp.zeros((16,), jnp.float32))
    acc_v[...] = acc * (0.25 / HALF)
    pltpu.sync_copy(acc_v, out_hbm.at[pl.ds(wid * 16, 16)])


def _sparse_corr_sc(T, means, D):
    t_flat = T.reshape(N * KPAD)
    mesh = plsc.VectorSubcoreMesh(core_axis_name="c", subcore_axis_name="s")
    out, _, _ = pl.kernel(
        _sc_body,
        out_type=(
            jax.ShapeDtypeStruct((_NW * 16,), jnp.float32),
            jax.ShapeDtypeStruct((_NC * N * KPAD,), jnp.int32),
            jax.ShapeDtypeStruct((_NC * N,), jnp.float32),
        ),
        mesh=mesh,
        compiler_params=pltpu.CompilerParams(needs_layout_passes=False),
        scratch_types=[
            pltpu.VMEM((N * KPAD,), jnp.int32),
            pltpu.VMEM((N * KPAD,), jnp.int32),
            pltpu.VMEM((N,), jnp.float32),
            pltpu.VMEM((N,), jnp.float32),
            pltpu.VMEM((2, N), jnp.float32),
            pltpu.VMEM((16,), jnp.float32),
            pltpu.SemaphoreType.DMA,
        ],
    )(t_flat, means, D)
    return jnp.sum(out)


def _sparse_corr_jnp(T, means, D):
    Tk = T[:, :TOPK]
    rows = jnp.arange(N)
    mut = (Tk[Tk] == rows[:, None, None]).any(-1)
    cnt = mut.sum(1).astype(jnp.float32)
    invc = 1.0 / jnp.maximum(cnt, 1.0)

    def g_pair(i, j):
        dij = D[i, j]
        sij = dij / means[i]
        sji = dij / means[j]
        gij = jnp.maximum(sij, 0.0) ** 2 - jnp.maximum(1.0 - sij, 0.0) ** 2
        gji = jnp.maximum(sji, 0.0) ** 2 - jnp.maximum(1.0 - sji, 0.0) ** 2
        return gij + gji

    r = Tk[:, :HALF]
    jmat = Tk[r]
    mut_r = mut[r]
    valid = mut_r & (jmat != rows[:, None, None])
    Tr = Tk[r]
    mutr = mut[r]
    Tj = Tk[jmat]
    mutj = mut[jmat]
    match = ((Tr[:, :, None, :, None] == Tj[:, :, :, None, :])
             & mutr[:, :, None, :, None] & mutj[:, :, :, None, :])
    M = match.sum((-1, -2)).astype(jnp.float32)
    coeff = M * invc[r][:, :, None]
    gp = g_pair(rows[:, None, None], jmat)
    corr = jnp.where(valid, coeff * gp, 0.0).sum()
    return 0.25 / HALF * corr


def kernel(s_emb, t_emb, idx):
    loss_parts, D, means, T = _dense_phase(s_emb, t_emb, idx)
    means = means.reshape(N)
    corr = _sparse_corr_sc(T, means, D)
    return (jnp.sum(loss_parts) + corr) / (N * (N - 1))


# speedup vs baseline: 1.6074x; 1.1618x over previous
import functools

import jax
import jax.numpy as jnp
from jax import lax
from jax.experimental import pallas as pl
from jax.experimental.pallas import tpu as pltpu
from jax.experimental.pallas import tpu_sc as plsc

N = 2048
DIM = 128
BLK = 256
NBLK = N // BLK
TOPK = 10
HALF = 5
KPAD = 17


def _dense_body(s_all_ref, s_blk_ref, t_all_ref, t_blk_ref, idx_ref,
                idx_blk_ref, loss_ref, d_ref, mean_ref, topk_ref):
    i = pl.program_id(0)
    s_all = s_all_ref[...]
    s_blk = s_blk_ref[...]
    ssq_all = jnp.sum(s_all * s_all, axis=1)
    ssq_blk = jnp.sum(s_blk * s_blk, axis=1)
    d2 = ssq_blk[:, None] + ssq_all[None, :] - 2.0 * jax.lax.dot_general(
        s_blk, s_all, (((1,), (1,)), ((), ())),
        preferred_element_type=jnp.float32)
    d2 = jnp.maximum(d2, 0.0)
    safe = jnp.where(d2 > 0.0, d2, 1.0)
    D = jnp.where(d2 > 0.0, jnp.sqrt(safe), 0.0)
    mean_i = jnp.mean(D, axis=1)
    S = D / mean_i[:, None]

    t_all = t_all_ref[...]
    t_blk = t_blk_ref[...]
    tn_all = t_all / jnp.maximum(
        jnp.sqrt(jnp.sum(t_all * t_all, axis=1, keepdims=True)), 1e-12)
    tn_blk = t_blk / jnp.maximum(
        jnp.sqrt(jnp.sum(t_blk * t_blk, axis=1, keepdims=True)), 1e-12)
    tsq_all = jnp.sum(tn_all * tn_all, axis=1)
    tsq_blk = jnp.sum(tn_blk * tn_blk, axis=1)
    td2 = tsq_blk[:, None] + tsq_all[None, :] - 2.0 * jax.lax.dot_general(
        tn_blk, tn_all, (((1,), (1,)), ((), ())),
        preferred_element_type=jnp.float32)
    td2 = jnp.maximum(td2, 0.0)
    tsafe = jnp.where(td2 > 0.0, td2, 1.0)
    Td = jnp.where(td2 > 0.0, jnp.sqrt(tsafe), 0.0)
    W_P = jnp.exp(-(Td * Td))

    idx_all = idx_ref[0, :]
    idx_blk = idx_blk_ref[0, :]
    same = idx_blk[:, None] == idx_all[None, :]
    wp = jnp.where(same, 1.0, W_P)

    col = lax.broadcasted_iota(jnp.int32, (BLK, N), 1)
    tk = []
    for _ in range(TOPK):
        m = jnp.max(wp, axis=1, keepdims=True)
        cand = jnp.min(jnp.where(wp >= m, col, N), axis=1)
        tk.append(cand)
        wp = jnp.where(col == cand[:, None], -jnp.inf, wp)
    tk_mat = jnp.stack(tk, axis=1)
    topk_ref[...] = jnp.pad(tk_mat, ((0, 0), (0, KPAD - TOPK)))

    base = jnp.maximum(1.0 - S, 0.0) ** 2
    g = jnp.maximum(S, 0.0) ** 2 - base
    row = lax.broadcasted_iota(jnp.int32, (BLK, N), 0)
    offdiag = (col != row + i * BLK).astype(jnp.float32)
    part = jnp.sum(offdiag * (base + 0.5 * W_P * g))
    lane = lax.broadcasted_iota(jnp.int32, (1, 1, 128), 2)
    loss_ref[...] = jnp.where(lane == 0, part, 0.0)

    d_ref[...] = D
    mean_ref[...] = mean_i[None, None, :]


def _dense_phase(s_emb, t_emb, idx):
    idx2 = idx.reshape(1, N)
    out = pl.pallas_call(
        _dense_body,
        grid=(NBLK,),
        in_specs=[
            pl.BlockSpec((N, DIM), lambda i: (0, 0)),
            pl.BlockSpec((BLK, DIM), lambda i: (i, 0)),
            pl.BlockSpec((N, DIM), lambda i: (0, 0)),
            pl.BlockSpec((BLK, DIM), lambda i: (i, 0)),
            pl.BlockSpec((1, N), lambda i: (0, 0)),
            pl.BlockSpec((1, BLK), lambda i: (0, i)),
        ],
        out_specs=[
            pl.BlockSpec((1, 1, 128), lambda i: (i, 0, 0)),
            pl.BlockSpec((BLK, N), lambda i: (i, 0)),
            pl.BlockSpec((1, 1, BLK), lambda i: (i, 0, 0)),
            pl.BlockSpec((BLK, KPAD), lambda i: (i, 0)),
        ],
        out_shape=[
            jax.ShapeDtypeStruct((NBLK, 1, 128), jnp.float32),
            jax.ShapeDtypeStruct((N, N), jnp.float32),
            jax.ShapeDtypeStruct((NBLK, 1, BLK), jnp.float32),
            jax.ShapeDtypeStruct((N, KPAD), jnp.int32),
        ],
    )(s_emb, s_emb, t_emb, t_emb, idx2, idx2)
    return out


_NC = 2
_NS = 16
_NW = _NC * _NS
_ROWS_PER_TEC = N // _NW
_ROWS_PER_SID = N // _NS


def _sc_body(t_hbm, means_hbm, d_hbm, out_hbm, mut_hbm, wc_hbm,
             T_v, mut_v, means_v, invc_v, drow_v, acc_v, sem):
    cid = lax.axis_index("c")
    sid = lax.axis_index("s")
    wid = sid * _NC + cid
    lanes = lax.iota(jnp.int32, 16)
    kvalid = lanes < TOPK
    zi = jnp.zeros((16,), jnp.int32)

    pltpu.sync_copy(t_hbm, T_v)
    pltpu.sync_copy(means_hbm, means_v)

    base_a = sid * _ROWS_PER_SID

    def mut_row(r, _):
        jv = plsc.load_gather(T_v, [r * KPAD + lanes])
        macc = jnp.zeros((16,), jnp.int32)
        for kp in range(TOPK):
            tv = plsc.load_gather(T_v, [jv * KPAD + kp], mask=kvalid)
            macc = macc | jnp.where((tv == r) & kvalid, 1, 0)
        plsc.store_scatter(mut_v, [r * KPAD + lanes], macc)
        cnt = jnp.sum(macc.astype(jnp.float32), axis=0)
        invv = 1.0 / jnp.maximum(jnp.zeros((16,), jnp.float32) + cnt, 1.0)
        plsc.store_scatter(invc_v, [zi + r], invv, mask=lanes == 0)
        return 0

    lax.fori_loop(base_a, base_a + _ROWS_PER_SID, mut_row, 0)

    off_a = base_a * KPAD
    sz_a = _ROWS_PER_SID * KPAD
    coff = cid * N * KPAD
    pltpu.sync_copy(mut_v.at[pl.ds(off_a, sz_a)],
                    mut_hbm.at[pl.ds(coff + off_a, sz_a)])
    pltpu.sync_copy(invc_v.at[pl.ds(base_a, _ROWS_PER_SID)],
                    wc_hbm.at[pl.ds(cid * N + base_a, _ROWS_PER_SID)])
    plsc.subcore_barrier()
    pltpu.sync_copy(mut_hbm.at[pl.ds(coff, N * KPAD)], mut_v)
    pltpu.sync_copy(wc_hbm.at[pl.ds(cid * N, N)], invc_v)

    base_b = wid * _ROWS_PER_TEC
    pltpu.async_copy(d_hbm.at[base_b], drow_v.at[0], sem)

    def row_i(i, acc):
        par = lax.rem(i - base_b, 2)
        pltpu.make_async_copy(d_hbm.at[base_b], drow_v.at[par], sem).wait()

        @pl.when(i + 1 < base_b + _ROWS_PER_TEC)
        def _prefetch():
            pltpu.async_copy(d_hbm.at[i + 1], drow_v.at[1 - par], sem)

        miv = plsc.load_gather(means_v, [zi + i])
        Trow = plsc.load_gather(T_v, [i * KPAD + lanes])
        for m in range(HALF):
            r = Trow[m]
            rvec = zi + r
            invr = plsc.load_gather(invc_v, [rvec])
            jv = plsc.load_gather(T_v, [rvec * KPAD + lanes])
            mutr = plsc.load_gather(mut_v, [rvec * KPAD + lanes])
            valid = kvalid & (mutr != 0) & (jv != i)
            Mv = jnp.zeros((16,), jnp.float32)
            for b in range(TOPK):
                tb = plsc.load_gather(T_v, [jv * KPAD + b], mask=valid)
                mb = plsc.load_gather(mut_v, [jv * KPAD + b], mask=valid)
                member = jnp.zeros((16,), jnp.int32)
                for a in range(TOPK):
                    hit = (tb == jv[a]) & (mutr[a] != 0)
                    member = member | jnp.where(hit, 1, 0)
                Mv = Mv + jnp.where((member != 0) & (mb != 0), 1.0, 0.0)
            dij = plsc.load_gather(drow_v, [zi + par, jv], mask=valid)
            mj = plsc.load_gather(means_v, [jv], mask=valid)
            sij = dij / miv
            sji = dij / jnp.where(valid, mj, 1.0)
            pij = jnp.maximum(sij, 0.0)
            qij = jnp.maximum(1.0 - sij, 0.0)
            pji = jnp.maximum(sji, 0.0)
            qji = jnp.maximum(1.0 - sji, 0.0)
            gij = pij * pij - qij * qij
            gji = pji * pji - qji * qji
            acc = acc + jnp.where(valid, Mv * invr * (gij + gji), 0.0)
        return acc

    acc = lax.fori_loop(base_b, base_b + _ROWS_PER_TEC, row_i,
                        jnp.zeros((16,), jnp.float32))
    acc_v[...] = acc * (0.25 / HALF)
    pltpu.sync_copy(acc_v, out_hbm.at[pl.ds(wid * 16, 16)])


def _sparse_corr_sc(T, means, D):
    t_flat = T.reshape(N * KPAD)
    mesh = plsc.VectorSubcoreMesh(core_axis_name="c", subcore_axis_name="s")
    out, _, _ = pl.kernel(
        _sc_body,
        out_type=(
            jax.ShapeDtypeStruct((_NW * 16,), jnp.float32),
            jax.ShapeDtypeStruct((_NC * N * KPAD,), jnp.int32),
            jax.ShapeDtypeStruct((_NC * N,), jnp.float32),
        ),
        mesh=mesh,
        compiler_params=pltpu.CompilerParams(needs_layout_passes=False),
        scratch_types=[
            pltpu.VMEM((N * KPAD,), jnp.int32),
            pltpu.VMEM((N * KPAD,), jnp.int32),
            pltpu.VMEM((N,), jnp.float32),
            pltpu.VMEM((N,), jnp.float32),
            pltpu.VMEM((2, N), jnp.float32),
            pltpu.VMEM((16,), jnp.float32),
            pltpu.SemaphoreType.DMA,
        ],
    )(t_flat, means, D)
    return jnp.sum(out)


def _sparse_corr_jnp(T, means, D):
    Tk = T[:, :TOPK]
    rows = jnp.arange(N)
    mut = (Tk[Tk] == rows[:, None, None]).any(-1)
    cnt = mut.sum(1).astype(jnp.float32)
    invc = 1.0 / jnp.maximum(cnt, 1.0)

    def g_pair(i, j):
        dij = D[i, j]
        sij = dij / means[i]
        sji = dij / means[j]
        gij = jnp.maximum(sij, 0.0) ** 2 - jnp.maximum(1.0 - sij, 0.0) ** 2
        gji = jnp.maximum(sji, 0.0) ** 2 - jnp.maximum(1.0 - sji, 0.0) ** 2
        return gij + gji

    r = Tk[:, :HALF]
    jmat = Tk[r]
    mut_r = mut[r]
    valid = mut_r & (jmat != rows[:, None, None])
    Tr = Tk[r]
    mutr = mut[r]
    Tj = Tk[jmat]
    mutj = mut[jmat]
    match = ((Tr[:, :, None, :, None] == Tj[:, :, :, None, :])
             & mutr[:, :, None, :, None] & mutj[:, :, :, None, :])
    M = match.sum((-1, -2)).astype(jnp.float32)
    coeff = M * invc[r][:, :, None]
    gp = g_pair(rows[:, None, None], jmat)
    corr = jnp.where(valid, coeff * gp, 0.0).sum()
    return 0.25 / HALF * corr


def kernel(s_emb, t_emb, idx):
    loss_parts, D, means, T = _dense_phase(s_emb, t_emb, idx)
    means = means.reshape(N)
    corr = _sparse_corr_sc(T, means, D)
    return (jnp.sum(loss_parts) + corr) / (N * (N - 1))
